# baseline (speedup 1.0000x reference)
import functools

import jax
import jax.numpy as jnp
from jax import lax
from jax.experimental import pallas as pl
from jax.experimental.pallas import tpu as pltpu

B, S, D = 64, 512, 2048
NUM_PROMPTS = 200
TOP_K = 64
NP_PAD = 256
ROWS = 64
NUM_J = 1 + S // ROWS


def _fused_body(cls_ref, prompts_ref, wq_ref, bq_ref, wp_ref, bp_ref, x_ref,
                out_ref, pproj_s, sim_s):
    b = pl.program_id(0)
    j = pl.program_id(1)

    @pl.when(jnp.logical_and(b == 0, j == 0))
    def _head():
        prompts = prompts_ref[...]
        pproj = lax.dot_general(prompts, wp_ref[...],
                                (((1,), (1,)), ((), ())),
                                preferred_element_type=jnp.float32)
        pproj = pproj + bp_ref[...]
        pproj_s[0:NUM_PROMPTS, :] = pproj
        pproj_s[NUM_PROMPTS:NP_PAD, :] = jnp.zeros(
            (NP_PAD - NUM_PROMPTS, D), jnp.float32)

        q = lax.dot_general(cls_ref[...], wq_ref[...],
                            (((1,), (1,)), ((), ())),
                            preferred_element_type=jnp.float32)
        q = q + bq_ref[...]
        qn = q * lax.rsqrt(jnp.maximum(
            jnp.sum(q * q, axis=1, keepdims=True), 1e-24))
        pnorm = jnp.sqrt(jnp.maximum(
            jnp.sum(prompts * prompts, axis=1, keepdims=True), 1e-24))
        sim = lax.dot_general(qn, prompts, (((1,), (1,)), ((), ())),
                              preferred_element_type=jnp.float32)
        sim = sim / pnorm.reshape(1, NUM_PROMPTS)
        sim_s[:, 0:NUM_PROMPTS] = sim
        sim_s[:, NUM_PROMPTS:NP_PAD] = jnp.full(
            (B, NP_PAD - NUM_PROMPTS), -2.0, jnp.float32)

    @pl.when(j == 0)
    def _select():
        srow = sim_s[pl.ds(b, 1), :]
        s_i = srow.reshape(NP_PAD, 1)
        s_j = srow
        ii = lax.broadcasted_iota(jnp.int32, (NP_PAD, NP_PAD), 0)
        jj = lax.broadcasted_iota(jnp.int32, (NP_PAD, NP_PAD), 1)
        beats = (s_j > s_i) | ((s_j == s_i) & (jj < ii))
        rank = jnp.sum(beats.astype(jnp.int32), axis=1)
        kk = lax.broadcasted_iota(jnp.int32, (TOP_K, NP_PAD), 0)
        onehot = (kk == rank.reshape(1, NP_PAD)).astype(jnp.float32)
        sel = lax.dot_general(onehot, pproj_s[...],
                              (((1,), (0,)), ((), ())),
                              preferred_element_type=jnp.float32)
        out_ref[0, :, :] = sel

    @pl.when(j > 0)
    def _copy():
        out_ref[0, :, :] = x_ref[0, :, :]


@jax.jit
def kernel(x, prompts_embeddings, Wq, bq, Wp, bp):
    cls = x[:, 0, :]
    bq2 = bq.reshape(1, D)
    bp2 = bp.reshape(1, D)

    grid = (B, NUM_J)
    out = pl.pallas_call(
        _fused_body,
        grid=grid,
        in_specs=[
            pl.BlockSpec((B, D), lambda b, j: (0, 0)),
            pl.BlockSpec((NUM_PROMPTS, D), lambda b, j: (0, 0)),
            pl.BlockSpec((D, D), lambda b, j: (0, 0)),
            pl.BlockSpec((1, D), lambda b, j: (0, 0)),
            pl.BlockSpec((D, D), lambda b, j: (0, 0)),
            pl.BlockSpec((1, D), lambda b, j: (0, 0)),
            pl.BlockSpec((1, ROWS, D),
                         lambda b, j: (b, jnp.maximum(j - 1, 0), 0)),
        ],
        out_specs=pl.BlockSpec((1, ROWS, D), lambda b, j: (b, j, 0)),
        out_shape=jax.ShapeDtypeStruct((B, TOP_K + S, D), jnp.float32),
        scratch_shapes=[
            pltpu.VMEM((NP_PAD, D), jnp.float32),
            pltpu.VMEM((B, NP_PAD), jnp.float32),
        ],
        compiler_params=pltpu.CompilerParams(
            dimension_semantics=("arbitrary", "arbitrary")),
    )(cls, prompts_embeddings, Wq, bq2, Wp, bp2, x)
    return out
